# baseline (speedup 1.0000x reference)
import functools

import jax
import jax.numpy as jnp
from jax import lax
from jax.experimental import pallas as pl
from jax.experimental.pallas import tpu as pltpu
from jax.experimental.pallas import tpu_sc as plsc

EMBED_DIM = 128
SEQLEN = 2048
BATCH = 4
NUM_CORES = 2
NUM_SUBCORES = 16
NUM_WORKERS = NUM_CORES * NUM_SUBCORES
WORKERS_PER_BATCH = NUM_WORKERS // BATCH
ROWS_PER_W = SEQLEN // WORKERS_PER_BATCH
CHUNK = 128
NCHUNK = ROWS_PER_W // CHUNK


@functools.partial(
    pl.kernel,
    mesh=plsc.VectorSubcoreMesh(core_axis_name="c", subcore_axis_name="s"),
    out_type=jax.ShapeDtypeStruct((BATCH, SEQLEN, EMBED_DIM), jnp.float32),
    scratch_types=[
        pltpu.VMEM((NCHUNK, CHUNK), jnp.int32),
        pltpu.VMEM((ROWS_PER_W, EMBED_DIM), jnp.float32),
        pltpu.SemaphoreType.DMA,
    ],
)
def _embed_kernel(ids_hbm, word_hbm, pos_hbm, out_hbm, idx_v, rows_v, sem):
    wid = lax.axis_index("s") * NUM_CORES + lax.axis_index("c")
    b = wid // WORKERS_PER_BATCH
    off = (wid % WORKERS_PER_BATCH) * ROWS_PER_W

    for j in range(NCHUNK):
        pltpu.sync_copy(ids_hbm.at[b, pl.ds(off + j * CHUNK, CHUNK)], idx_v.at[j])
    pltpu.sync_copy(pos_hbm.at[pl.ds(off, ROWS_PER_W)], rows_v)

    copies = [
        pltpu.async_copy(
            word_hbm.at[idx_v.at[j]],
            rows_v.at[pl.ds(j * CHUNK, CHUNK)],
            sem,
            add=True,
        )
        for j in range(NCHUNK)
    ]
    for cp in copies:
        cp.wait()

    pltpu.sync_copy(rows_v, out_hbm.at[b, pl.ds(off, ROWS_PER_W)])


def kernel(input_ids, word_table, pos_table):
    return _embed_kernel(input_ids.astype(jnp.int32), word_table, pos_table)
